# baseline (speedup 1.0000x reference)
import functools

import jax
import jax.numpy as jnp
from jax import lax
from jax.experimental import pallas as pl
from jax.experimental.pallas import tpu as pltpu
from jax.experimental.pallas import tpu_sc as plsc

_NC = 2
_NS = 16
_NW = _NC * _NS
_L = 16
_CH = 128


@jax.jit
def _gather_relu(table, idx):
    V, D = table.shape
    (B,) = idx.shape
    b_per_w = B // _NW
    nchunk = b_per_w // _CH

    mesh = plsc.VectorSubcoreMesh(core_axis_name="c", subcore_axis_name="s")

    @functools.partial(
        pl.kernel,
        mesh=mesh,
        out_type=jax.ShapeDtypeStruct((B, D), jnp.float32),
        scratch_types=[
            pltpu.VMEM((b_per_w,), jnp.int32),
            pltpu.VMEM((b_per_w, D), jnp.float32),
            pltpu.SemaphoreType.DMA((nchunk,)),
            pltpu.SemaphoreType.DMA,
        ],
    )
    def k(table_hbm, idx_hbm, out_hbm, idx_v, rows_v, gsem, ssem):
        wid = lax.axis_index("s") * _NC + lax.axis_index("c")
        base = wid * b_per_w
        pltpu.sync_copy(idx_hbm.at[pl.ds(base, b_per_w)], idx_v)

        gathers = [
            pltpu.async_copy(
                table_hbm.at[idx_v.at[pl.ds(c * _CH, _CH)]],
                rows_v.at[pl.ds(c * _CH, _CH)],
                gsem.at[c],
            )
            for c in range(nchunk)
        ]
        for c in range(nchunk):
            gathers[c].wait()

            @plsc.parallel_loop(c * _CH, (c + 1) * _CH)
            def relu_rows(r):
                for j in range(D // _L):
                    x = rows_v[r, pl.ds(j * _L, _L)]
                    rows_v[r, pl.ds(j * _L, _L)] = jnp.maximum(x, 0.0)

            pltpu.async_copy(
                rows_v.at[pl.ds(c * _CH, _CH)],
                out_hbm.at[pl.ds(base + c * _CH, _CH)],
                ssem,
            )
        pltpu.make_async_copy(rows_v, out_hbm.at[pl.ds(base, b_per_w)], ssem).wait()

    return k(table, idx)


def kernel(assignments, edge_index, node_idx):
    del edge_index
    return _gather_relu(assignments, node_idx.astype(jnp.int32))


# speedup vs baseline: 1.0016x; 1.0016x over previous
import functools

import jax
import jax.numpy as jnp
from jax import lax
from jax.experimental import pallas as pl
from jax.experimental.pallas import tpu as pltpu
from jax.experimental.pallas import tpu_sc as plsc

_NC = 2
_NS = 16
_NW = _NC * _NS
_L = 16
_CH = 64


@jax.jit
def _gather_relu(table, idx):
    V, D = table.shape
    (B,) = idx.shape
    b_per_w = B // _NW
    nchunk = b_per_w // _CH

    mesh = plsc.VectorSubcoreMesh(core_axis_name="c", subcore_axis_name="s")

    @functools.partial(
        pl.kernel,
        mesh=mesh,
        out_type=jax.ShapeDtypeStruct((B, D), jnp.float32),
        scratch_types=[
            pltpu.VMEM((b_per_w,), jnp.int32),
            pltpu.VMEM((b_per_w, D), jnp.float32),
            pltpu.SemaphoreType.DMA((nchunk,)),
            pltpu.SemaphoreType.DMA,
        ],
    )
    def k(table_hbm, idx_hbm, out_hbm, idx_v, rows_v, gsem, ssem):
        wid = lax.axis_index("s") * _NC + lax.axis_index("c")
        base = wid * b_per_w
        pltpu.sync_copy(idx_hbm.at[pl.ds(base, b_per_w)], idx_v)

        gathers = [
            pltpu.async_copy(
                table_hbm.at[idx_v.at[pl.ds(c * _CH, _CH)]],
                rows_v.at[pl.ds(c * _CH, _CH)],
                gsem.at[c],
            )
            for c in range(nchunk)
        ]
        for c in range(nchunk):
            gathers[c].wait()

            @plsc.parallel_loop(c * _CH, (c + 1) * _CH)
            def relu_rows(r):
                for j in range(D // _L):
                    x = rows_v[r, pl.ds(j * _L, _L)]
                    rows_v[r, pl.ds(j * _L, _L)] = jnp.maximum(x, 0.0)

            pltpu.async_copy(
                rows_v.at[pl.ds(c * _CH, _CH)],
                out_hbm.at[pl.ds(base + c * _CH, _CH)],
                ssem,
            )
        pltpu.make_async_copy(rows_v, out_hbm.at[pl.ds(base, b_per_w)], ssem).wait()

    return k(table, idx)


def kernel(assignments, edge_index, node_idx):
    del edge_index
    return _gather_relu(assignments, node_idx.astype(jnp.int32))
